# baseline (speedup 1.0000x reference)
import functools

import jax
import jax.numpy as jnp
from jax import lax
from jax.experimental import pallas as pl
from jax.experimental.pallas import tpu as pltpu
from jax.experimental.pallas import tpu_sc as plsc

EMB_DIM = 64
BATCH = 16384
GLOBAL_TIME_NUM = 128
MAX_GLOBAL_TIME = 86400.0
_INV_SLOT_W = GLOBAL_TIME_NUM / MAX_GLOBAL_TIME

NC = 2
NS = 16
L = 16
NW = NC * NS
RPW = BATCH // NW
HALF = RPW // 2
CH = 128


def _build():
    mesh = plsc.VectorSubcoreMesh(core_axis_name="c", subcore_axis_name="s")

    @functools.partial(
        pl.kernel,
        out_type=jax.ShapeDtypeStruct((BATCH, 3 * EMB_DIM), jnp.float32),
        mesh=mesh,
        compiler_params=pltpu.CompilerParams(use_tc_tiling_on_sc=False),
        scratch_types=[
            pltpu.VMEM((RPW,), jnp.int32),
            pltpu.VMEM((RPW,), jnp.int32),
            pltpu.VMEM((RPW,), jnp.int32),
            pltpu.VMEM((RPW,), jnp.float32),
            pltpu.VMEM((RPW,), jnp.int32),
            pltpu.VMEM((HALF, EMB_DIM), jnp.float32),
            pltpu.VMEM((HALF, EMB_DIM), jnp.float32),
            pltpu.VMEM((HALF, EMB_DIM), jnp.float32),
            pltpu.VMEM((HALF, EMB_DIM), jnp.float32),
            pltpu.SemaphoreType.DMA,
        ],
    )
    def emb_kernel(cas_i, src_i, dst_i, times, usrc, udst, cstate, ttab, out,
                   srci, dsti, casi, timv, slotv, srow, drow, crow, trow, sem):
        wid = lax.axis_index("s") * NC + lax.axis_index("c")
        base = wid * RPW
        pltpu.sync_copy(src_i.at[pl.ds(base, RPW)], srci)
        pltpu.sync_copy(dst_i.at[pl.ds(base, RPW)], dsti)
        pltpu.sync_copy(cas_i.at[pl.ds(base, RPW)], casi)
        pltpu.sync_copy(times.at[pl.ds(base, RPW)], timv)

        def slot_body(j, carry):
            t = timv[pl.ds(j * L, L)]
            s = jnp.clip((t * _INV_SLOT_W).astype(jnp.int32), 0, GLOBAL_TIME_NUM - 1)
            slotv[pl.ds(j * L, L)] = s
            return carry

        lax.fori_loop(0, RPW // L, slot_body, 0)

        for h in range(2):
            rb = h * HALF
            cps = []
            for cnk in range(2):
                o = rb + cnk * CH
                d = pl.ds(cnk * CH, CH)
                cps.append(pltpu.async_copy(usrc.at[srci.at[pl.ds(o, CH)]], srow.at[d], sem))
                cps.append(pltpu.async_copy(udst.at[dsti.at[pl.ds(o, CH)]], drow.at[d], sem))
                cps.append(pltpu.async_copy(cstate.at[casi.at[pl.ds(o, CH)]], crow.at[d], sem))
                cps.append(pltpu.async_copy(ttab.at[slotv.at[pl.ds(o, CH)]], trow.at[d], sem))
            for cp in cps:
                cp.wait()

            def add_body(r, carry):
                for c in range(EMB_DIM // L):
                    sl = pl.ds(c * L, L)
                    crow[r, sl] = crow[r, sl] + trow[r, sl]
                return carry

            lax.fori_loop(0, HALF, add_body, 0)

            gb = base + rb
            pltpu.sync_copy(srow, out.at[pl.ds(gb, HALF), pl.ds(0, EMB_DIM)])
            pltpu.sync_copy(drow, out.at[pl.ds(gb, HALF), pl.ds(EMB_DIM, EMB_DIM)])
            pltpu.sync_copy(crow, out.at[pl.ds(gb, HALF), pl.ds(2 * EMB_DIM, EMB_DIM)])

    return emb_kernel


_emb = _build()


def kernel(cascades, src_idx, dst_idx, cas_pub_times, user_src_state,
           user_dst_state, cas_state, time_table):
    return _emb(cascades.astype(jnp.int32), src_idx.astype(jnp.int32),
                dst_idx.astype(jnp.int32), cas_pub_times,
                user_src_state, user_dst_state, cas_state, time_table)


# speedup vs baseline: 1.0043x; 1.0043x over previous
import functools

import jax
import jax.numpy as jnp
from jax import lax
from jax.experimental import pallas as pl
from jax.experimental.pallas import tpu as pltpu
from jax.experimental.pallas import tpu_sc as plsc

EMB_DIM = 64
BATCH = 16384
GLOBAL_TIME_NUM = 128
MAX_GLOBAL_TIME = 86400.0
_INV_SLOT_W = GLOBAL_TIME_NUM / MAX_GLOBAL_TIME

NC = 2
NS = 16
L = 16
NW = NC * NS
RPW = BATCH // NW
CH = 128
NCH = RPW // CH


def _build():
    mesh = plsc.VectorSubcoreMesh(core_axis_name="c", subcore_axis_name="s")

    @functools.partial(
        pl.kernel,
        out_type=jax.ShapeDtypeStruct((BATCH, 3 * EMB_DIM), jnp.float32),
        mesh=mesh,
        compiler_params=pltpu.CompilerParams(use_tc_tiling_on_sc=False),
        scratch_types=[
            pltpu.VMEM((RPW,), jnp.int32),
            pltpu.VMEM((RPW,), jnp.int32),
            pltpu.VMEM((RPW,), jnp.int32),
            pltpu.VMEM((RPW,), jnp.float32),
            pltpu.VMEM((RPW,), jnp.int32),
            pltpu.VMEM((RPW, EMB_DIM), jnp.float32),
            pltpu.VMEM((RPW, EMB_DIM), jnp.float32),
            pltpu.VMEM((RPW, EMB_DIM), jnp.float32),
            pltpu.VMEM((CH, EMB_DIM), jnp.float32),
            pltpu.VMEM((CH, EMB_DIM), jnp.float32),
            pltpu.SemaphoreType.DMA,
            pltpu.SemaphoreType.DMA,
            pltpu.SemaphoreType.DMA,
            pltpu.SemaphoreType.DMA,
            pltpu.SemaphoreType.DMA,
            pltpu.SemaphoreType.DMA,
        ],
    )
    def emb_kernel(cas_i, src_i, dst_i, times, usrc, udst, cstate, ttab, out,
                   srci, dsti, casi, timv, slotv, srow, drow, crow, trowa, trowb,
                   sem_i, sem_s, sem_d, sem_c, sem_t, sem_o):
        wid = lax.axis_index("s") * NC + lax.axis_index("c")
        base = wid * RPW
        ci = [
            pltpu.async_copy(src_i.at[pl.ds(base, RPW)], srci, sem_i),
            pltpu.async_copy(dst_i.at[pl.ds(base, RPW)], dsti, sem_i),
            pltpu.async_copy(cas_i.at[pl.ds(base, RPW)], casi, sem_i),
            pltpu.async_copy(times.at[pl.ds(base, RPW)], timv, sem_i),
        ]
        for c in ci:
            c.wait()

        def chunk(k):
            return pl.ds(k * CH, CH)

        gs = [pltpu.async_copy(usrc.at[srci.at[chunk(k)]], srow.at[chunk(k)], sem_s)
              for k in range(NCH)]
        gd = [pltpu.async_copy(udst.at[dsti.at[chunk(k)]], drow.at[chunk(k)], sem_d)
              for k in range(NCH)]
        gc = [pltpu.async_copy(cstate.at[casi.at[chunk(k)]], crow.at[chunk(k)], sem_c)
              for k in range(NCH)]

        def slot_body(j, carry):
            t = timv[pl.ds(j * L, L)]
            s = jnp.clip((t * _INV_SLOT_W).astype(jnp.int32), 0, GLOBAL_TIME_NUM - 1)
            slotv[pl.ds(j * L, L)] = s
            return carry

        lax.fori_loop(0, RPW // L, slot_body, 0)

        tbufs = [trowa, trowb]
        gt = [pltpu.async_copy(ttab.at[slotv.at[chunk(k)]], tbufs[k % 2], sem_t)
              for k in range(2)]

        for g in gs:
            g.wait()
        outs = [pltpu.async_copy(srow, out.at[pl.ds(base, RPW), pl.ds(0, EMB_DIM)], sem_o)]
        for g in gd:
            g.wait()
        outs.append(pltpu.async_copy(drow, out.at[pl.ds(base, RPW), pl.ds(EMB_DIM, EMB_DIM)], sem_o))
        for g in gc:
            g.wait()

        def add_chunk(k, tbuf):
            def add_body(r, carry):
                for c in range(EMB_DIM // L):
                    sl = pl.ds(c * L, L)
                    rr = k * CH + r
                    crow[rr, sl] = crow[rr, sl] + tbuf[r, sl]
                return carry
            lax.fori_loop(0, CH, add_body, 0)

        for k in range(NCH):
            gt[k].wait()
            add_chunk(k, tbufs[k % 2])
            if k + 2 < NCH:
                gt.append(pltpu.async_copy(ttab.at[slotv.at[chunk(k + 2)]],
                                           tbufs[k % 2], sem_t))
        outs.append(pltpu.async_copy(crow, out.at[pl.ds(base, RPW), pl.ds(2 * EMB_DIM, EMB_DIM)], sem_o))
        for o in outs:
            o.wait()

    return emb_kernel


_emb = _build()


def kernel(cascades, src_idx, dst_idx, cas_pub_times, user_src_state,
           user_dst_state, cas_state, time_table):
    return _emb(cascades.astype(jnp.int32), src_idx.astype(jnp.int32),
                dst_idx.astype(jnp.int32), cas_pub_times,
                user_src_state, user_dst_state, cas_state, time_table)


# speedup vs baseline: 1.3446x; 1.3389x over previous
import functools

import jax
import jax.numpy as jnp
from jax import lax
from jax.experimental import pallas as pl
from jax.experimental.pallas import tpu as pltpu
from jax.experimental.pallas import tpu_sc as plsc

EMB_DIM = 64
BATCH = 16384
GLOBAL_TIME_NUM = 128
MAX_GLOBAL_TIME = 86400.0
_INV_SLOT_W = GLOBAL_TIME_NUM / MAX_GLOBAL_TIME

N_USERS = 1000000
N_CAS = 100000

NC = 2
NS = 16
L = 16
NW = NC * NS
RPW = BATCH // NW
NB = RPW // L
QTR = RPW // 4


def _build():
    mesh = plsc.VectorSubcoreMesh(core_axis_name="c", subcore_axis_name="s")

    @functools.partial(
        pl.kernel,
        out_type=tuple(jax.ShapeDtypeStruct((BATCH // 8, 8, EMB_DIM), jnp.float32)
                       for _ in range(3)),
        mesh=mesh,
        compiler_params=pltpu.CompilerParams(use_tc_tiling_on_sc=True),
        scratch_types=[
            pltpu.VMEM((RPW,), jnp.int32),
            pltpu.VMEM((RPW,), jnp.int32),
            pltpu.VMEM((RPW,), jnp.int32),
            pltpu.VMEM((RPW,), jnp.float32),
            pltpu.VMEM((RPW,), jnp.int32),
            pltpu.VMEM((GLOBAL_TIME_NUM, EMB_DIM), jnp.float32),
            pltpu.VMEM((L, 8, EMB_DIM), jnp.float32),
            pltpu.VMEM((L, 8, EMB_DIM), jnp.float32),
            pltpu.VMEM((L, 8, EMB_DIM), jnp.float32),
            pltpu.VMEM((QTR, EMB_DIM), jnp.float32),
            pltpu.VMEM((QTR, EMB_DIM), jnp.float32),
            pltpu.VMEM((QTR, EMB_DIM), jnp.float32),
            pltpu.SemaphoreType.DMA,
            pltpu.SemaphoreType.DMA,
            pltpu.SemaphoreType.DMA,
            pltpu.SemaphoreType.DMA,
            pltpu.SemaphoreType.DMA,
            pltpu.SemaphoreType.DMA,
            pltpu.SemaphoreType.DMA,
        ],
    )
    def emb_kernel(cas_i, src_i, dst_i, times, usrc, udst, cstate, ttab,
                   out_s, out_d, out_c,
                   srci, dsti, casi, timv, slotv, ttabv, sgrp, dgrp, cgrp,
                   sstg, dstg, cstg,
                   sem_i, sem_s, sem_d, sem_c, sem_os, sem_od, sem_oc):
        wid = lax.axis_index("s") * NC + lax.axis_index("c")
        base = wid * RPW
        ci = [
            pltpu.async_copy(src_i.at[pl.ds(base, RPW)], srci, sem_i),
            pltpu.async_copy(dst_i.at[pl.ds(base, RPW)], dsti, sem_i),
            pltpu.async_copy(cas_i.at[pl.ds(base, RPW)], casi, sem_i),
            pltpu.async_copy(times.at[pl.ds(base, RPW)], timv, sem_i),
        ]
        pltpu.sync_copy(ttab, ttabv)
        for c in ci:
            c.wait()

        usrc3 = usrc.reshape(N_USERS // 8, 8, EMB_DIM)
        udst3 = udst.reshape(N_USERS // 8, 8, EMB_DIM)
        cst3 = cstate.reshape(N_CAS // 8, 8, EMB_DIM)

        def slot_body(j, carry):
            t = timv[pl.ds(j * L, L)]
            s = jnp.clip((t * _INV_SLOT_W).astype(jnp.int32), 0, GLOBAL_TIME_NUM - 1)
            slotv[pl.ds(j * L, L)] = s
            return carry

        lax.fori_loop(0, NB, slot_body, 0)

        def fire_batch(idxref, row0, table2d, grp, sem):
            v = idxref[pl.ds(row0, L)]
            gv = v >> 3
            cps = []
            for t in range(L):
                g = pl.multiple_of(gv[t] * 8, 8)
                cps.append(pltpu.async_copy(table2d.at[pl.ds(g, 8)], grp.at[t], sem))
            return cps

        def drain_batch(table3d, grp, sem):
            pltpu.make_async_copy(table3d.at[pl.ds(0, L)], grp, sem).wait()

        def quarter_loop(h):
            def body(b, carry):
                row0 = h * QTR + b * L
                fire_batch(srci, row0, usrc, sgrp, sem_s)
                fire_batch(dsti, row0, udst, dgrp, sem_d)
                fire_batch(casi, row0, cstate, cgrp, sem_c)
                sv = srci[pl.ds(row0, L)] & 7
                dv = dsti[pl.ds(row0, L)] & 7
                cv = casi[pl.ds(row0, L)] & 7
                tv = slotv[pl.ds(row0, L)]
                drain_batch(usrc3, sgrp, sem_s)
                drain_batch(udst3, dgrp, sem_d)
                drain_batch(cst3, cgrp, sem_c)
                srow0 = b * L
                for t in range(L):
                    rs = sv[t]
                    rd = dv[t]
                    rc = cv[t]
                    ts = tv[t]
                    for c in range(EMB_DIM // L):
                        sl = pl.ds(c * L, L)
                        sstg[srow0 + t, sl] = sgrp[t, rs, sl]
                        dstg[srow0 + t, sl] = dgrp[t, rd, sl]
                        cstg[srow0 + t, sl] = cgrp[t, rc, sl] + ttabv[ts, sl]
                return carry

            lax.fori_loop(0, NB // 4, body, 0)

        for h in range(4):
            if h >= 1:
                pltpu.make_async_copy(out_s.at[pl.ds(0, QTR // 8)], sstg.reshape(QTR // 8, 8, EMB_DIM), sem_os).wait()
                pltpu.make_async_copy(out_d.at[pl.ds(0, QTR // 8)], dstg.reshape(QTR // 8, 8, EMB_DIM), sem_od).wait()
                pltpu.make_async_copy(out_c.at[pl.ds(0, QTR // 8)], cstg.reshape(QTR // 8, 8, EMB_DIM), sem_oc).wait()
            quarter_loop(h)
            g0 = (base + h * QTR) // 8
            pltpu.async_copy(sstg.reshape(QTR // 8, 8, EMB_DIM), out_s.at[pl.ds(g0, QTR // 8)], sem_os)
            pltpu.async_copy(dstg.reshape(QTR // 8, 8, EMB_DIM), out_d.at[pl.ds(g0, QTR // 8)], sem_od)
            pltpu.async_copy(cstg.reshape(QTR // 8, 8, EMB_DIM), out_c.at[pl.ds(g0, QTR // 8)], sem_oc)

        pltpu.make_async_copy(out_s.at[pl.ds(0, QTR // 8)], sstg.reshape(QTR // 8, 8, EMB_DIM), sem_os).wait()
        pltpu.make_async_copy(out_d.at[pl.ds(0, QTR // 8)], dstg.reshape(QTR // 8, 8, EMB_DIM), sem_od).wait()
        pltpu.make_async_copy(out_c.at[pl.ds(0, QTR // 8)], cstg.reshape(QTR // 8, 8, EMB_DIM), sem_oc).wait()

    return emb_kernel


_emb = _build()


def kernel(cascades, src_idx, dst_idx, cas_pub_times, user_src_state,
           user_dst_state, cas_state, time_table):
    s, d, c = _emb(cascades.astype(jnp.int32), src_idx.astype(jnp.int32),
                   dst_idx.astype(jnp.int32), cas_pub_times,
                   user_src_state, user_dst_state, cas_state, time_table)
    return jnp.concatenate([s.reshape(BATCH, EMB_DIM),
                            d.reshape(BATCH, EMB_DIM),
                            c.reshape(BATCH, EMB_DIM)], axis=1)
